# baseline (speedup 1.0000x reference)
import functools

import jax
import jax.numpy as jnp
from jax.experimental import pallas as pl
from jax.experimental.pallas import tpu as pltpu


def _bdot(a, b):
    return jnp.dot(a.astype(jnp.bfloat16), b.astype(jnp.bfloat16),
                   preferred_element_type=jnp.float32)


def _branch_body(adj_ref, meth_ref, w1_ref, fc1w_ref, w2_ref,
                 b1_ref, b2_ref, fc1b_ref, pool_ref,
                 abf_ref, xb_ref, su_ref, init_ref, *, n, np_, tm, n_tiles):
    t = pl.program_id(1)

    @pl.when(t == 0)
    def _proj():
        xb = meth_ref[0].astype(jnp.bfloat16)
        xb_ref[...] = xb
        init_ref[...] = (
            jnp.dot(xb, fc1w_ref[...].astype(jnp.bfloat16),
                    preferred_element_type=jnp.float32)
            + fc1b_ref[...]
        )

    a_t = adj_ref[0]
    abf_ref[pl.ds(t * tm, tm), :] = a_t
    t1 = jnp.dot(a_t, xb_ref[...], preferred_element_type=jnp.float32)
    su_t = jnp.maximum(
        jnp.dot(t1.astype(jnp.bfloat16), w1_ref[...].astype(jnp.bfloat16),
                preferred_element_type=jnp.float32)
        + b1_ref[...],
        0.0,
    )
    su_ref[pl.ds(t * tm, tm), :] = su_t.astype(jnp.bfloat16)

    @pl.when(t == n_tiles - 1)
    def _pass2():
        su = su_ref[...]
        w2 = w2_ref[...].astype(jnp.bfloat16)
        b2 = b2_ref[...]
        m = None
        for t0 in range(0, np_, tm):
            a2 = abf_ref[t0:t0 + tm, :]
            t2 = jnp.dot(a2, su, preferred_element_type=jnp.float32)
            o = (
                jnp.dot(t2.astype(jnp.bfloat16), w2,
                        preferred_element_type=jnp.float32)
                + init_ref[t0:t0 + tm, :]
                + b2
            )
            if t0 + tm > n:
                rows = t0 + jax.lax.broadcasted_iota(jnp.int32, o.shape, 0)
                o = jnp.where(rows < n, o, -jnp.inf)
            tmax = jnp.max(o, axis=0, keepdims=True)
            m = tmax if m is None else jnp.maximum(m, tmax)
        pool_ref[0] = m


def _branch(adj_p, meth_p, w1, b1, w2, b2, fc1w, fc1b, n):
    bsz, np_, f = meth_p.shape
    h = w1.shape[1]
    c = w2.shape[1]
    tm = 544
    n_tiles = np_ // tm

    pool = pl.pallas_call(
        functools.partial(_branch_body, n=n, np_=np_, tm=tm,
                          n_tiles=n_tiles),
        grid=(bsz, n_tiles),
        in_specs=[
            pl.BlockSpec((1, tm, np_), lambda b, t: (b, t, 0)),
            pl.BlockSpec((1, np_, f), lambda b, t: (b, 0, 0)),
            pl.BlockSpec((f, h), lambda b, t: (0, 0)),
            pl.BlockSpec((f, c), lambda b, t: (0, 0)),
            pl.BlockSpec((h, c), lambda b, t: (0, 0)),
            pl.BlockSpec((1, h), lambda b, t: (0, 0)),
            pl.BlockSpec((1, c), lambda b, t: (0, 0)),
            pl.BlockSpec((1, c), lambda b, t: (0, 0)),
        ],
        out_specs=pl.BlockSpec((1, 1, c), lambda b, t: (b, 0, 0)),
        out_shape=jax.ShapeDtypeStruct((bsz, 1, c), jnp.float32),
        scratch_shapes=[
            pltpu.VMEM((np_, np_), jnp.bfloat16),
            pltpu.VMEM((np_, f), jnp.bfloat16),
            pltpu.VMEM((np_, h), jnp.bfloat16),
            pltpu.VMEM((np_, c), jnp.float32),
        ],
        compiler_params=pltpu.CompilerParams(
            dimension_semantics=("arbitrary", "arbitrary"),
            vmem_limit_bytes=64 * 1024 * 1024,
        ),
    )(adj_p, meth_p, w1, fc1w, w2, b1, b2, fc1b)

    return pool


def _mlp_body(sp_ref, vp_ref, w2a_ref, w2b_ref, b2_ref, w3_ref, b3_ref,
              w4_ref, b4_ref, w5_ref, b5_ref, out_ref):
    d = jnp.maximum(
        _bdot(sp_ref[:, 0, :], w2a_ref[...])
        + _bdot(vp_ref[:, 0, :], w2b_ref[...])
        + b2_ref[...],
        0.0,
    )
    d = jnp.maximum(_bdot(d, w3_ref[...]) + b3_ref[...], 0.0)
    d = jnp.maximum(_bdot(d, w4_ref[...]) + b4_ref[...], 0.0)
    w5 = w5_ref[...].astype(jnp.bfloat16).astype(jnp.float32)
    db = d.astype(jnp.bfloat16).astype(jnp.float32)
    out_ref[...] = jnp.sum(db * w5.T, axis=1, keepdims=True) + b5_ref[...]


def kernel(solute_adj, solute_meth, solvent_meth, solvent_adj_meth,
           conv1_W, conv1_b, conv2_W, conv2_b,
           fc1_W, fc1_b, fc2_W, fc2_b, fc3_W, fc3_b,
           fc4_W, fc4_b, fc5_W, fc5_b):
    b1 = conv1_b.reshape(1, -1)
    b2 = conv2_b.reshape(1, -1)
    fb1 = fc1_b.reshape(1, -1)
    nclass = fc1_W.shape[1]
    n = solute_meth.shape[1]
    np_ = 2176
    pad = np_ - n

    su_adj = jnp.pad(solute_adj.astype(jnp.bfloat16),
                     ((0, 0), (0, pad), (0, pad)))
    sv_adj = jnp.pad(solvent_adj_meth.astype(jnp.bfloat16),
                     ((0, 0), (0, pad), (0, pad)))
    su_meth = jnp.pad(solute_meth, ((0, 0), (0, pad), (0, 0)))
    sv_meth = jnp.pad(solvent_meth, ((0, 0), (0, pad), (0, 0)))

    su_pool = _branch(su_adj, su_meth, conv1_W, b1, conv2_W, b2,
                      fc1_W, fb1, n)
    sv_pool = _branch(sv_adj, sv_meth, conv1_W, b1, conv2_W,
                      b2, fc1_W, fb1, n)

    bsz = su_pool.shape[0]
    out = pl.pallas_call(
        _mlp_body,
        out_shape=jax.ShapeDtypeStruct((bsz, 1), jnp.float32),
    )(su_pool, sv_pool,
      fc2_W[:nclass], fc2_W[nclass:], fc2_b.reshape(1, -1),
      fc3_W, fc3_b.reshape(1, -1),
      fc4_W, fc4_b.reshape(1, -1),
      fc5_W, fc5_b.reshape(1, -1))
    return out


# speedup vs baseline: 1.3020x; 1.3020x over previous
import functools

import jax
import jax.numpy as jnp
from jax.experimental import pallas as pl
from jax.experimental.pallas import tpu as pltpu


def _bdot(a, b):
    return jnp.dot(a.astype(jnp.bfloat16), b.astype(jnp.bfloat16),
                   preferred_element_type=jnp.float32)


def _branch_body(adj_ref, meth_ref, w1_ref, fc1w_ref, w2_ref,
                 b1_ref, b2_ref, fc1b_ref, pool_ref,
                 abf_ref, xb_ref, su_ref, init_ref, *, n, tm, n_tiles):
    t = pl.program_id(1)

    @pl.when(t == 0)
    def _proj():
        xb = meth_ref[0].astype(jnp.bfloat16)
        xb_ref[...] = xb
        init_ref[0:n, :] = (
            jnp.dot(xb, fc1w_ref[...].astype(jnp.bfloat16),
                    preferred_element_type=jnp.float32)
            + fc1b_ref[...]
        )

    a_t = adj_ref[0].astype(jnp.bfloat16)
    abf_ref[pl.ds(t * tm, tm), :] = a_t
    t1 = jnp.dot(a_t, xb_ref[...], preferred_element_type=jnp.float32)
    su_t = jnp.maximum(
        jnp.dot(t1.astype(jnp.bfloat16), w1_ref[...].astype(jnp.bfloat16),
                preferred_element_type=jnp.float32)
        + b1_ref[...],
        0.0,
    )
    su_ref[pl.ds(t * tm, tm), :] = su_t.astype(jnp.bfloat16)

    @pl.when(t == n_tiles - 1)
    def _pass2():
        su = su_ref[0:n, :]
        w2 = w2_ref[...].astype(jnp.bfloat16)
        b2 = b2_ref[...]
        m = None
        for t0 in range(0, n_tiles * tm, tm):
            a2 = abf_ref[t0:t0 + tm, :]
            t2 = jnp.dot(a2, su, preferred_element_type=jnp.float32)
            o = (
                jnp.dot(t2.astype(jnp.bfloat16), w2,
                        preferred_element_type=jnp.float32)
                + init_ref[t0:t0 + tm, :]
                + b2
            )
            if t0 + tm > n:
                rows = t0 + jax.lax.broadcasted_iota(jnp.int32, o.shape, 0)
                o = jnp.where(rows < n, o, -jnp.inf)
            tmax = jnp.max(o, axis=0, keepdims=True)
            m = tmax if m is None else jnp.maximum(m, tmax)
        pool_ref[0] = m


def _branch(adj, meth, w1, b1, w2, b2, fc1w, fc1b):
    bsz, n, f = meth.shape
    h = w1.shape[1]
    c = w2.shape[1]
    tm = 528
    n_tiles = -(-n // tm)
    n_pad = n_tiles * tm

    pool = pl.pallas_call(
        functools.partial(_branch_body, n=n, tm=tm, n_tiles=n_tiles),
        grid=(bsz, n_tiles),
        in_specs=[
            pl.BlockSpec((1, tm, n), lambda b, t: (b, t, 0)),
            pl.BlockSpec((1, n, f), lambda b, t: (b, 0, 0)),
            pl.BlockSpec((f, h), lambda b, t: (0, 0)),
            pl.BlockSpec((f, c), lambda b, t: (0, 0)),
            pl.BlockSpec((h, c), lambda b, t: (0, 0)),
            pl.BlockSpec((1, h), lambda b, t: (0, 0)),
            pl.BlockSpec((1, c), lambda b, t: (0, 0)),
            pl.BlockSpec((1, c), lambda b, t: (0, 0)),
        ],
        out_specs=pl.BlockSpec((1, 1, c), lambda b, t: (b, 0, 0)),
        out_shape=jax.ShapeDtypeStruct((bsz, 1, c), jnp.float32),
        scratch_shapes=[
            pltpu.VMEM((n_pad, n), jnp.bfloat16),
            pltpu.VMEM((n, f), jnp.bfloat16),
            pltpu.VMEM((n_pad, h), jnp.bfloat16),
            pltpu.VMEM((n_pad, c), jnp.float32),
        ],
        compiler_params=pltpu.CompilerParams(
            dimension_semantics=("arbitrary", "arbitrary"),
            vmem_limit_bytes=64 * 1024 * 1024,
        ),
    )(adj, meth, w1, fc1w, w2, b1, b2, fc1b)

    return pool


def _mlp_body(sp_ref, vp_ref, w2a_ref, w2b_ref, b2_ref, w3_ref, b3_ref,
              w4_ref, b4_ref, w5_ref, b5_ref, out_ref):
    d = jnp.maximum(
        _bdot(sp_ref[:, 0, :], w2a_ref[...])
        + _bdot(vp_ref[:, 0, :], w2b_ref[...])
        + b2_ref[...],
        0.0,
    )
    d = jnp.maximum(_bdot(d, w3_ref[...]) + b3_ref[...], 0.0)
    d = jnp.maximum(_bdot(d, w4_ref[...]) + b4_ref[...], 0.0)
    w5 = w5_ref[...].astype(jnp.bfloat16).astype(jnp.float32)
    db = d.astype(jnp.bfloat16).astype(jnp.float32)
    out_ref[...] = jnp.sum(db * w5.T, axis=1, keepdims=True) + b5_ref[...]


def kernel(solute_adj, solute_meth, solvent_meth, solvent_adj_meth,
           conv1_W, conv1_b, conv2_W, conv2_b,
           fc1_W, fc1_b, fc2_W, fc2_b, fc3_W, fc3_b,
           fc4_W, fc4_b, fc5_W, fc5_b):
    b1 = conv1_b.reshape(1, -1)
    b2 = conv2_b.reshape(1, -1)
    fb1 = fc1_b.reshape(1, -1)
    nclass = fc1_W.shape[1]

    su_pool = _branch(solute_adj, solute_meth, conv1_W, b1, conv2_W, b2,
                      fc1_W, fb1)
    sv_pool = _branch(solvent_adj_meth, solvent_meth, conv1_W, b1, conv2_W,
                      b2, fc1_W, fb1)

    bsz = su_pool.shape[0]
    out = pl.pallas_call(
        _mlp_body,
        out_shape=jax.ShapeDtypeStruct((bsz, 1), jnp.float32),
    )(su_pool, sv_pool,
      fc2_W[:nclass], fc2_W[nclass:], fc2_b.reshape(1, -1),
      fc3_W, fc3_b.reshape(1, -1),
      fc4_W, fc4_b.reshape(1, -1),
      fc5_W, fc5_b.reshape(1, -1))
    return out
